# Initial kernel scaffold; baseline (speedup 1.0000x reference)
import jax
import jax.numpy as jnp
from jax.experimental import pallas as pl


def kernel(anchor, positive, negative):
    raise NotImplementedError("write your pallas kernel here")


# speedup vs baseline: 1.4208x; 1.4208x over previous
import jax
import jax.numpy as jnp
from jax.experimental import pallas as pl
from jax.experimental.pallas import tpu as pltpu

_MARGIN = 1.0
_EPS = 1e-6
_BLK = 512


def _triplet_loss_kernel(a_ref, p_ref, n_ref, out_ref):
    A = a_ref[:, :]
    R, D = A.shape
    C = n_ref.shape[0]

    a2 = jnp.sum(A * A, axis=1, keepdims=True)
    sa = jnp.sum(A, axis=1, keepdims=True)

    ones_row = jnp.ones((1, D), dtype=jnp.float32)

    def body(b, carry):
        best_key, best_dn2 = carry
        nb = n_ref[pl.ds(b * _BLK, _BLK), :]
        n2b = jax.lax.dot_general(ones_row, nb * nb, (((1,), (1,)), ((), ())),
                                  preferred_element_type=jnp.float32)
        snb = jax.lax.dot_general(ones_row, nb, (((1,), (1,)), ((), ())),
                                  preferred_element_type=jnp.float32)
        g = jax.lax.dot_general(A, nb, (((1,), (1,)), ((), ())),
                                preferred_element_type=jnp.float32)
        d2 = a2 + n2b - 2.0 * g
        key = jnp.maximum(d2, 0.0)
        mk = jnp.min(key, axis=1, keepdims=True)
        dn2m = d2 + (2.0 * _EPS) * (sa - snb)
        picked = jnp.min(jnp.where(key == mk, dn2m, jnp.inf),
                         axis=1, keepdims=True)
        better = mk < best_key
        return (jnp.where(better, mk, best_key),
                jnp.where(better, picked, best_dn2))

    init = (jnp.full((R, 1), jnp.inf, dtype=jnp.float32),
            jnp.zeros((R, 1), dtype=jnp.float32))
    _, best_dn2 = jax.lax.fori_loop(0, C // _BLK, body, init)

    dn = jnp.sqrt(jnp.maximum(best_dn2 + D * _EPS * _EPS, 0.0))
    diff = A - p_ref[:, :] + _EPS
    dp = jnp.sqrt(jnp.sum(diff * diff, axis=1, keepdims=True))
    losses = jnp.maximum(dp - dn + _MARGIN, 0.0)
    out_ref[:, :] = jnp.sum(losses, axis=0, keepdims=True) / R


def kernel(anchor, positive, negative):
    out = pl.pallas_call(
        _triplet_loss_kernel,
        out_shape=jax.ShapeDtypeStruct((1, 1), jnp.float32),
    )(anchor, positive, negative)
    return out[0, 0]


# speedup vs baseline: 3.2510x; 2.2882x over previous
import jax
import jax.numpy as jnp
from jax.experimental import pallas as pl
from jax.experimental.pallas import tpu as pltpu

_MARGIN = 1.0
_EPS = 1e-6
_BLK = 1024


def _triplet_loss_kernel(a_ref, p_ref, n_ref, out_ref):
    A = a_ref[:, :]
    R, D = A.shape
    N = n_ref[:, :]
    C = N.shape[0]

    a2 = jnp.sum(A * A, axis=1, keepdims=True)
    sa = jnp.sum(A, axis=1, keepdims=True)
    row_term = a2 + (2.0 * _EPS) * sa

    n2 = jnp.sum(N * N, axis=1, keepdims=True)
    sn = jnp.sum(N, axis=1, keepdims=True)
    n_aug = jnp.concatenate([N * -2.0, n2 - (2.0 * _EPS) * sn], axis=1)
    a_aug = jnp.concatenate([A, jnp.ones((R, 1), jnp.float32)], axis=1)

    best = jnp.full((R, 1), jnp.inf, dtype=jnp.float32)
    for b in range(C // _BLK):
        nb = jax.lax.slice(n_aug, (b * _BLK, 0), ((b + 1) * _BLK, D + 1))
        z = jax.lax.dot_general(a_aug, nb, (((1,), (1,)), ((), ())),
                                preferred_element_type=jnp.float32)
        best = jnp.minimum(best, jnp.min(row_term + z, axis=1, keepdims=True))

    dn = jnp.sqrt(jnp.maximum(best + D * _EPS * _EPS, 0.0))
    diff = A - p_ref[:, :] + _EPS
    dp = jnp.sqrt(jnp.sum(diff * diff, axis=1, keepdims=True))
    losses = jnp.maximum(dp - dn + _MARGIN, 0.0)
    out_ref[:, :] = jnp.sum(losses, axis=0, keepdims=True) / R


def kernel(anchor, positive, negative):
    out = pl.pallas_call(
        _triplet_loss_kernel,
        out_shape=jax.ShapeDtypeStruct((1, 1), jnp.float32),
    )(anchor, positive, negative)
    return out[0, 0]


# speedup vs baseline: 3.4235x; 1.0531x over previous
import jax
import jax.numpy as jnp
from jax.experimental import pallas as pl
from jax.experimental.pallas import tpu as pltpu

_MARGIN = 1.0
_EPS = 1e-6
_BLK = 1024


def _triplet_loss_kernel(a_ref, p_ref, n_ref, out_ref):
    A = a_ref[:, :]
    R, D = A.shape
    N = n_ref[:, :]
    C = N.shape[0]

    row_term = jnp.sum(A * A + (2.0 * _EPS) * A, axis=1, keepdims=True)
    col_term = jnp.sum(N * N - (2.0 * _EPS) * N, axis=1, keepdims=True)
    ones_r = jnp.ones((R, 1), dtype=jnp.float32)
    ones_c = jnp.ones((C, 1), dtype=jnp.float32)
    a_aug = jnp.concatenate([A, ones_r, row_term], axis=1)
    n_aug = jnp.concatenate([N * -2.0, col_term, ones_c], axis=1)

    best = jnp.full((R, 1), jnp.inf, dtype=jnp.float32)
    for b in range(C // _BLK):
        nb = jax.lax.slice(n_aug, (b * _BLK, 0), ((b + 1) * _BLK, D + 2))
        z = jax.lax.dot_general(a_aug, nb, (((1,), (1,)), ((), ())),
                                preferred_element_type=jnp.float32)
        best = jnp.minimum(best, jnp.min(z, axis=1, keepdims=True))

    dn = jnp.sqrt(jnp.maximum(best + D * _EPS * _EPS, 0.0))
    diff = A - p_ref[:, :] + _EPS
    dp = jnp.sqrt(jnp.sum(diff * diff, axis=1, keepdims=True))
    losses = jnp.maximum(dp - dn + _MARGIN, 0.0)
    out_ref[:, :] = jnp.sum(losses, axis=0, keepdims=True) / R


def kernel(anchor, positive, negative):
    out = pl.pallas_call(
        _triplet_loss_kernel,
        out_shape=jax.ShapeDtypeStruct((1, 1), jnp.float32),
    )(anchor, positive, negative)
    return out[0, 0]
